# baseline (speedup 1.0000x reference)
import jax
import jax.numpy as jnp
from jax.experimental import pallas as pl
from jax.experimental.pallas import tpu as pltpu

_BLOCK_ROWS = 400


def _support_body(x_ref, w_ref, s_ref):
    s_ref[...] = jnp.dot(x_ref[...], w_ref[...],
                         preferred_element_type=jnp.float32)


def _agg_body(adj_ref, s_ref, b_ref, out_ref):
    out_ref[...] = jnp.dot(adj_ref[...], s_ref[...],
                           preferred_element_type=jnp.float32) + b_ref[...]


def kernel(infeatn, adj, weight, bias):
    n, din = infeatn.shape
    dout = weight.shape[1]
    bias2d = bias.reshape(1, dout)

    support = pl.pallas_call(
        _support_body,
        out_shape=jax.ShapeDtypeStruct((n, dout), jnp.float32),
    )(infeatn, weight)

    bi = _BLOCK_ROWS
    grid = (n // bi,)
    out = pl.pallas_call(
        _agg_body,
        grid=grid,
        in_specs=[
            pl.BlockSpec((bi, n), lambda i: (i, 0)),
            pl.BlockSpec((n, dout), lambda i: (0, 0)),
            pl.BlockSpec((1, dout), lambda i: (0, 0)),
        ],
        out_specs=pl.BlockSpec((bi, dout), lambda i: (i, 0)),
        out_shape=jax.ShapeDtypeStruct((n, dout), jnp.float32),
        compiler_params=pltpu.CompilerParams(
            dimension_semantics=("parallel",),
        ),
    )(adj, support, bias2d)
    return out


# speedup vs baseline: 1.0434x; 1.0434x over previous
import jax
import jax.numpy as jnp
from jax.experimental import pallas as pl
from jax.experimental.pallas import tpu as pltpu

_BLOCK_ROWS = 400


def _gcn_body(x_ref, w_ref, adj_ref, b_ref, out_ref, s_ref):
    @pl.when(pl.program_id(0) == 0)
    def _():
        s_ref[...] = jnp.dot(x_ref[...], w_ref[...],
                             preferred_element_type=jnp.float32)

    out_ref[...] = jnp.dot(adj_ref[...], s_ref[...],
                           preferred_element_type=jnp.float32) + b_ref[...]


def kernel(infeatn, adj, weight, bias):
    n, din = infeatn.shape
    dout = weight.shape[1]
    bias2d = bias.reshape(1, dout)

    bi = _BLOCK_ROWS
    grid = (n // bi,)
    out = pl.pallas_call(
        _gcn_body,
        grid=grid,
        in_specs=[
            pl.BlockSpec((n, din), lambda i: (0, 0)),
            pl.BlockSpec((din, dout), lambda i: (0, 0)),
            pl.BlockSpec((bi, n), lambda i: (i, 0)),
            pl.BlockSpec((1, dout), lambda i: (0, 0)),
        ],
        out_specs=pl.BlockSpec((bi, dout), lambda i: (i, 0)),
        out_shape=jax.ShapeDtypeStruct((n, dout), jnp.float32),
        scratch_shapes=[pltpu.VMEM((n, dout), jnp.float32)],
        compiler_params=pltpu.CompilerParams(
            dimension_semantics=("arbitrary",),
        ),
    )(infeatn, weight, adj, bias2d)
    return out
